# baseline (speedup 1.0000x reference)
import functools

import jax
import jax.numpy as jnp
from jax import lax
from jax.experimental import pallas as pl
from jax.experimental.pallas import tpu as pltpu
from jax.experimental.pallas import tpu_sc as plsc

D = 64
G = 16384
NC = 2
NS = 16
NW = NC * NS
BPW = G // NW
NCH = 4
CH = BPW // NCH
L = 16


def _build_sc_kernel():
    mesh = plsc.VectorSubcoreMesh(core_axis_name="c", subcore_axis_name="s")

    @functools.partial(
        pl.kernel,
        mesh=mesh,
        compiler_params=pltpu.CompilerParams(use_tc_tiling_on_sc=False),
        out_type=jax.ShapeDtypeStruct((NW, NCH, CH, D), jnp.float32),
        scratch_types=[
            pltpu.VMEM((NCH, CH), jnp.int32),
            pltpu.VMEM((NCH, CH), jnp.int32),
            pltpu.VMEM((NCH, CH), jnp.int32),
            pltpu.VMEM((NCH, CH, D), jnp.float32),
            pltpu.VMEM((NCH, CH, D), jnp.float32),
            pltpu.VMEM((NCH, CH, D), jnp.float32),
            pltpu.SemaphoreType.DMA,
        ],
    )
    def k(idx_hbm, midx_hbm, mask_hbm, pe_hbm, mt_hbm, sel_hbm, out_hbm,
          idx_v, midx_v, mask_v, rows_pe, rows_m, rows_k, sem):
        wid = lax.axis_index("s") * NC + lax.axis_index("c")

        pltpu.sync_copy(idx_hbm.at[wid], idx_v)
        pltpu.sync_copy(midx_hbm.at[wid], midx_v)
        pltpu.sync_copy(mask_hbm.at[wid], mask_v)

        copies = []
        for c in range(NCH):
            copies.append(
                pltpu.async_copy(pe_hbm.at[idx_v.at[c]], rows_pe.at[c], sem))
            copies.append(
                pltpu.async_copy(mt_hbm.at[midx_v.at[c]], rows_m.at[c], sem))
            copies.append(
                pltpu.async_copy(sel_hbm.at[mask_v.at[c]], rows_k.at[c], sem))
        for cp in copies:
            cp.wait()

        def body(r, carry):
            for c in range(NCH):
                for j in range(D // L):
                    sl = pl.ds(L * j, L)
                    pe = rows_pe[c, r, sl]
                    ms = rows_m[c, r, sl]
                    mk = rows_k[c, r, sl]
                    rows_pe[c, r, sl] = mk * pe + ms
            return carry

        lax.fori_loop(0, CH, body, 0)

        pltpu.sync_copy(rows_pe, out_hbm.at[wid])

    return k


@jax.jit
def kernel(indices, present_mask, missing_idx_map, pe_table, missing_table):
    idx = jnp.where(present_mask, indices, 0).astype(jnp.int32).reshape(NW, NCH, CH)
    n_missing = missing_table.shape[0]
    midx = jnp.where(present_mask, n_missing, missing_idx_map)
    midx = midx.astype(jnp.int32).reshape(NW, NCH, CH)
    mask = present_mask.astype(jnp.int32).reshape(NW, NCH, CH)
    mt_ext = jnp.concatenate(
        [missing_table.astype(jnp.float32),
         jnp.zeros((7, D), jnp.float32)], axis=0)
    sel_table = jnp.concatenate(
        [jnp.zeros((1, D), jnp.float32), jnp.ones((7, D), jnp.float32)], axis=0)
    out = _build_sc_kernel()(idx, midx, mask,
                             pe_table.astype(jnp.float32),
                             mt_ext, sel_table)
    return out.reshape(G, D)


# speedup vs baseline: 3.4044x; 3.4044x over previous
import functools

import jax
import jax.numpy as jnp
from jax import lax
from jax.experimental import pallas as pl
from jax.experimental.pallas import tpu as pltpu
from jax.experimental.pallas import tpu_sc as plsc

D = 64
G = 16384
NC = 2
NS = 16
NW = NC * NS
BPW = G // NW
L = 16
TR = 8
NGRP = BPW // L


def _build_sc_kernel():
    mesh = plsc.VectorSubcoreMesh(core_axis_name="c", subcore_axis_name="s")

    @functools.partial(
        pl.kernel,
        mesh=mesh,
        out_type=jax.ShapeDtypeStruct((NW, BPW, D), jnp.float32),
        scratch_types=[
            pltpu.VMEM((BPW,), jnp.int32),
            pltpu.VMEM((BPW,), jnp.int32),
            pltpu.VMEM((BPW,), jnp.int32),
            pltpu.VMEM((BPW,), jnp.float32),
            pltpu.VMEM((2, L, TR, D), jnp.float32),
            pltpu.VMEM((2, L, 128), jnp.float32),
            pltpu.VMEM((L, D), jnp.float32),
            pltpu.SemaphoreType.DMA,
            pltpu.SemaphoreType.DMA,
            pltpu.SemaphoreType.DMA,
            pltpu.SemaphoreType.DMA,
        ],
    )
    def k(tidx_hbm, sub_hbm, midx_hbm, mask_hbm, pe_hbm, mt_hbm, out_hbm,
          tidx_v, sub_v, midx_v, mask_v, blk_v, ms_v, out_v,
          semp0, semp1, semm0, semm1):
        wid = lax.axis_index("s") * NC + lax.axis_index("c")
        semp = (semp0, semp1)
        semm = (semm0, semm1)

        pltpu.sync_copy(tidx_hbm.at[wid], tidx_v)
        pltpu.sync_copy(sub_hbm.at[wid], sub_v)
        pltpu.sync_copy(midx_hbm.at[wid], midx_v)
        pltpu.sync_copy(mask_hbm.at[wid], mask_v)

        def fire(g, slot):
            tvec = tidx_v[pl.ds(g * L, L)]
            for k in range(L):
                pltpu.async_copy(pe_hbm.at[tvec[k]], blk_v.at[slot, k],
                                 semp[slot])
            pltpu.async_copy(mt_hbm.at[midx_v.at[pl.ds(g * L, L)]],
                             ms_v.at[slot], semm[slot])

        def wait(slot):
            for k in range(L):
                pltpu.make_async_copy(pe_hbm.at[0], blk_v.at[slot, k],
                                      semp[slot]).wait()
            pltpu.make_async_copy(mt_hbm.at[midx_v.at[pl.ds(0, L)]],
                                  ms_v.at[slot], semm[slot]).wait()

        def extract(g, slot):
            rvec = sub_v[pl.ds(g * L, L)]
            mvec = mask_v[pl.ds(g * L, L)]
            for k in range(L):
                r = rvec[k]
                m = mvec[k]
                for j in range(D // L):
                    sl = pl.ds(L * j, L)
                    pe = blk_v[slot, k, r, sl]
                    ms = ms_v[slot, k, sl]
                    out_v[k, sl] = m * (pe - ms) + ms
            pltpu.sync_copy(out_v, out_hbm.at[wid, pl.ds(g * L, L)])

        fire(0, 0)

        def pair(p, carry):
            ga = 2 * p
            fire(ga + 1, 1)
            wait(0)
            extract(ga, 0)

            @pl.when(p + 1 < NGRP // 2)
            def _():
                fire(ga + 2, 0)

            wait(1)
            extract(ga + 1, 1)
            return carry

        lax.fori_loop(0, NGRP // 2, pair, 0)

    return k


@jax.jit
def kernel(indices, present_mask, missing_idx_map, pe_table, missing_table):
    idx = indices.astype(jnp.int32)
    tidx = (idx // TR).reshape(NW, BPW)
    sub = (idx % TR).reshape(NW, BPW)
    midx = missing_idx_map.astype(jnp.int32).reshape(NW, BPW)
    mask = present_mask.astype(jnp.float32).reshape(NW, BPW)
    n_missing = missing_table.shape[0]
    mt_ext = jnp.zeros((n_missing, 128), jnp.float32)
    mt_ext = lax.dynamic_update_slice(
        mt_ext, missing_table.astype(jnp.float32), (0, 0))
    pe3 = pe_table.astype(jnp.float32).reshape(pe_table.shape[0] // TR, TR, D)
    out = _build_sc_kernel()(tidx, sub, midx, mask, pe3, mt_ext)
    return out.reshape(G, D)
